# baseline (speedup 1.0000x reference)
import functools

import jax
import jax.numpy as jnp
from jax import lax
from jax.experimental import pallas as pl
from jax.experimental.pallas import tpu as pltpu
from jax.experimental.pallas import tpu_sc as plsc

N = 10000
E = 160000
D = 256

NPAD = 10240
EPAD = 160000
NC = 2
NS = 16
NW = NC * NS
RPW = NPAD // NW
TRASH = RPW
ACC_ROWS = RPW + 8
ACC_WORDS = ACC_ROWS * D
OUT_WORDS = RPW * D

SCC = 2000
NSC = EPAD // SCC
GB = 32
MAXC = 4352
DUMP = MAXC - 1
DRAIN_AT = 2048
PSHIFT = 14
PMASK = (1 << PSHIFT) - 1


def _mm_body(x_ref, w_ref, o_ref):
    o_ref[...] = jnp.dot(x_ref[...], w_ref[...],
                         preferred_element_type=jnp.float32)


def _matmul(x_pad, w):
    return pl.pallas_call(
        _mm_body,
        grid=(NPAD // 1024,),
        in_specs=[pl.BlockSpec((1024, D), lambda i: (i, 0)),
                  pl.BlockSpec((D, D), lambda i: (0, 0))],
        out_specs=pl.BlockSpec((1024, D), lambda i: (i, 0)),
        out_shape=jax.ShapeDtypeStruct((NPAD, D), jnp.float32),
    )(x_pad, w)


@functools.partial(
    pl.kernel,
    mesh=plsc.VectorSubcoreMesh(core_axis_name="c", subcore_axis_name="s"),
    out_type=jax.ShapeDtypeStruct((NPAD * D,), jnp.float32),
    compiler_params=pltpu.CompilerParams(needs_layout_passes=False),
    scratch_types=[
        pltpu.VMEM((ACC_WORDS,), jnp.float32),
        pltpu.VMEM((SCC,), jnp.int32),
        pltpu.VMEM((SCC,), jnp.int32),
        pltpu.VMEM((MAXC,), jnp.int32),
        pltpu.VMEM((GB,), jnp.int32),
        pltpu.VMEM((GB,), jnp.int32),
        pltpu.VMEM((GB,), jnp.int32),
        pltpu.VMEM((GB,), jnp.int32),
        pltpu.VMEM((GB, D // 2), jnp.int32),
        pltpu.VMEM((GB, D // 2), jnp.int32),
        pltpu.SemaphoreType.DMA,
        pltpu.SemaphoreType.DMA,
        pltpu.SemaphoreType.DMA,
    ],
)
def _sc_agg(out_hbm, edges_hbm, zeros_hbm, agg_hbm,
            acc, es0, es1, comp_p,
            gc0, gc1, gl0, gl1, buf0, buf1,
            sem0, sem1, sems):
    c = lax.axis_index("c")
    s = lax.axis_index("s")
    wid = s * NC + c
    lo = wid * RPW

    pltpu.sync_copy(zeros_hbm, acc)

    trash_p = jnp.full((16,), TRASH << PSHIFT, jnp.int32)
    zero_v = jnp.zeros((16,), jnp.int32)
    one_v = jnp.ones((16,), jnp.int32)
    dump_v = jnp.full((16,), DUMP, jnp.int32)
    iota16 = lax.iota(jnp.int32, 16)
    pm_v = jnp.full((16,), PMASK, jnp.int32)
    psh_v = jnp.full((16,), PSHIFT, jnp.int32)
    losh_v = jnp.full((16,), lo << PSHIFT, jnp.int32)
    hish_v = jnp.full((16,), (lo + RPW) << PSHIFT, jnp.int32)

    def g_start(g, gcols, glidx, buf, sem):
        goff = pl.multiple_of(g * GB, GB)
        for q in range(GB // 16):
            qo = pl.multiple_of(q * 16, 16)
            p = comp_p[pl.ds(goff + qo, 16)]
            gcols[pl.ds(qo, 16)] = p & pm_v
            glidx[pl.ds(qo, 16)] = lax.shift_right_logical(p, psh_v)
        pltpu.make_async_copy(out_hbm.at[gcols], buf, sem).start()

    def g_wait(gcols, buf, sem):
        pltpu.make_async_copy(out_hbm.at[gcols], buf, sem).wait()

    def accum(glidx, buf):
        for g16 in range(GB // 16):
            lv = glidx[pl.ds(pl.multiple_of(g16 * 16, 16), 16)]
            for i in range(16):
                li = lv[i]
                ab = pl.multiple_of(li * D, 16)
                bi = g16 * 16 + i
                for q in range(D // 32):
                    qo = pl.multiple_of(q * 16, 16)
                    v32 = buf[bi, pl.ds(qo, 16)]
                    pab = plsc.bitcast(v32, jnp.bfloat16)
                    va, vb = plsc.unpack(
                        pab, format=plsc.PackFormat.INTERLEAVED)
                    co = pl.multiple_of(q * 32, 32)
                    plsc.addupdate(acc.at[pl.ds(ab + co, 16)], va)
                    plsc.addupdate(acc.at[pl.ds(ab + co + 16, 16)], vb)

    def drain(cnt):
        for p in range(2):
            ppos = jnp.full((16,), cnt + p * 16, jnp.int32) + iota16
            plsc.store_scatter(comp_p, [ppos], trash_p)
        nb = (cnt + GB - 1) // GB

        @pl.when(nb > 0)
        def _():
            g_start(0, gc0, gl0, buf0, sem0)

        def pair(g2, carry):
            b0 = g2 * 2
            b1 = b0 + 1

            @pl.when(b0 < nb)
            def _():
                g_wait(gc0, buf0, sem0)

                @pl.when(b1 < nb)
                def _():
                    g_start(b1, gc1, gl1, buf1, sem1)

                accum(gl0, buf0)

            @pl.when(b1 < nb)
            def _():
                g_wait(gc1, buf1, sem1)

                @pl.when(b1 + 1 < nb)
                def _():
                    g_start(b1 + 1, gc0, gl0, buf0, sem0)

                accum(gl1, buf1)

            return carry

        lax.fori_loop(0, (nb + 1) // 2, pair, 0)
        return 0

    def e_start(k, es, sem):
        koff = pl.multiple_of(k * SCC, SCC)
        pltpu.make_async_copy(
            edges_hbm.at[pl.ds(koff, SCC)], es, sem).start()

    def e_wait(k, es, sem):
        koff = pl.multiple_of(k * SCC, SCC)
        pltpu.make_async_copy(
            edges_hbm.at[pl.ds(koff, SCC)], es, sem).wait()

    def scan(es, ccs):
        def vec(i, ccs):
            jj = pl.multiple_of(i * 16, 16)
            p = es[pl.ds(jj, 16)]
            m = (p >= losh_v) & (p < hish_v)
            pcv = plsc.all_reduce_population_count(m)
            incl = plsc.cumsum(jnp.where(m, one_v, zero_v))
            pos = jnp.where(m, ccs + incl - one_v, dump_v)
            plsc.store_scatter(comp_p, [pos], p - losh_v)
            return ccs + pcv

        return lax.fori_loop(0, SCC // 16, vec, ccs, unroll=4)

    e_start(0, es0, sems)
    ccs0 = jnp.zeros((16,), jnp.int32)

    def chunk(k, ccs):
        even = k % 2 == 0

        @pl.when(even)
        def _():
            e_wait(k, es0, sems)

        @pl.when(~even)
        def _():
            e_wait(k, es1, sems)

        @pl.when(k + 1 < NSC)
        def _():
            @pl.when(even)
            def _():
                e_start(k + 1, es1, sems)

            @pl.when(~even)
            def _():
                e_start(k + 1, es0, sems)

        ccs = lax.cond(
            even,
            lambda cc: scan(es0, cc),
            lambda cc: scan(es1, cc),
            ccs,
        )
        cnt = ccs[0]
        cnt = lax.cond(
            (cnt >= DRAIN_AT) | (k == NSC - 1), drain, lambda cc: cc, cnt)
        return jnp.full((16,), cnt, jnp.int32)

    lax.fori_loop(0, NSC, chunk, ccs0)

    pltpu.sync_copy(acc.at[pl.ds(0, OUT_WORDS)],
                    agg_hbm.at[pl.ds(lo * D, OUT_WORDS)])


def _perm():
    idx = []
    for q in range(D // 32):
        for t in range(16):
            idx.append(q * 32 + t)
            idx.append(q * 32 + 16 + t)
    return idx


_PERM = tuple(_perm())


def kernel(x, edge_index, W, b):
    x_pad = jnp.concatenate(
        [x, jnp.zeros((NPAD - N, D), x.dtype)], axis=0)
    out = _matmul(x_pad, W)
    out_bf = out.astype(jnp.bfloat16)[:, jnp.array(_PERM, jnp.int32)]
    out32 = jax.lax.bitcast_convert_type(
        out_bf.reshape(NPAD, D // 2, 2), jnp.int32)
    edges = (edge_index[0] << PSHIFT) | edge_index[1]
    zeros = jnp.zeros((ACC_WORDS,), jnp.float32)
    agg = _sc_agg(out32, edges, zeros)
    return agg.reshape(NPAD, D)[:N] + b


# speedup vs baseline: 1.1192x; 1.1192x over previous
import functools

import jax
import jax.numpy as jnp
from jax import lax
from jax.experimental import pallas as pl
from jax.experimental.pallas import tpu as pltpu
from jax.experimental.pallas import tpu_sc as plsc

N = 10000
E = 160000
D = 256

NPAD = 10240
EPAD = 160000
NC = 2
NS = 16
NW = NC * NS
RPW = NPAD // NW
TRASH = RPW
ACC_ROWS = RPW + 8
ACC_WORDS = ACC_ROWS * D
OUT_WORDS = RPW * D

SCC = 2000
NSC = EPAD // SCC
GB = 16
MAXC = 4352
DUMP = MAXC - 1
DRAIN_AT = 2048
PSHIFT = 14
PMASK = (1 << PSHIFT) - 1


def _mm_body(x_ref, w_ref, o_ref):
    o_ref[...] = jnp.dot(x_ref[...], w_ref[...],
                         preferred_element_type=jnp.float32)


def _matmul(x_pad, w):
    return pl.pallas_call(
        _mm_body,
        grid=(NPAD // 1024,),
        in_specs=[pl.BlockSpec((1024, D), lambda i: (i, 0)),
                  pl.BlockSpec((D, D), lambda i: (0, 0))],
        out_specs=pl.BlockSpec((1024, D), lambda i: (i, 0)),
        out_shape=jax.ShapeDtypeStruct((NPAD, D), jnp.float32),
    )(x_pad, w)


@functools.partial(
    pl.kernel,
    mesh=plsc.VectorSubcoreMesh(core_axis_name="c", subcore_axis_name="s"),
    out_type=jax.ShapeDtypeStruct((NPAD * D,), jnp.float32),
    compiler_params=pltpu.CompilerParams(needs_layout_passes=False),
    scratch_types=[
        pltpu.VMEM((ACC_WORDS,), jnp.float32),
        pltpu.VMEM((SCC,), jnp.int32),
        pltpu.VMEM((SCC,), jnp.int32),
        pltpu.VMEM((MAXC,), jnp.int32),
        pltpu.VMEM((GB,), jnp.int32),
        pltpu.VMEM((GB,), jnp.int32),
        pltpu.VMEM((GB,), jnp.int32),
        pltpu.VMEM((GB,), jnp.int32),
        pltpu.VMEM((GB, D // 2), jnp.int32),
        pltpu.VMEM((GB, D // 2), jnp.int32),
        pltpu.SemaphoreType.DMA,
        pltpu.SemaphoreType.DMA,
        pltpu.SemaphoreType.DMA,
    ],
)
def _sc_agg(out_hbm, edges_hbm, zeros_hbm, agg_hbm,
            acc, es0, es1, comp_p,
            gc0, gc1, gl0, gl1, buf0, buf1,
            sem0, sem1, sems):
    c = lax.axis_index("c")
    s = lax.axis_index("s")
    wid = s * NC + c
    lo = wid * RPW

    pltpu.sync_copy(zeros_hbm, acc)

    trash_p = jnp.full((16,), TRASH << PSHIFT, jnp.int32)
    zero_v = jnp.zeros((16,), jnp.int32)
    one_v = jnp.ones((16,), jnp.int32)
    dump_v = jnp.full((16,), DUMP, jnp.int32)
    iota16 = lax.iota(jnp.int32, 16)
    pm_v = jnp.full((16,), PMASK, jnp.int32)
    psh_v = jnp.full((16,), PSHIFT, jnp.int32)
    losh_v = jnp.full((16,), lo << PSHIFT, jnp.int32)
    hish_v = jnp.full((16,), (lo + RPW) << PSHIFT, jnp.int32)

    def g_start(g, gcols, glidx, buf, sem):
        goff = pl.multiple_of(g * GB, GB)
        for q in range(GB // 16):
            qo = pl.multiple_of(q * 16, 16)
            p = comp_p[pl.ds(goff + qo, 16)]
            gcols[pl.ds(qo, 16)] = p & pm_v
            glidx[pl.ds(qo, 16)] = lax.shift_right_logical(p, psh_v)
        pltpu.make_async_copy(out_hbm.at[gcols], buf, sem).start()

    def g_wait(gcols, buf, sem):
        pltpu.make_async_copy(out_hbm.at[gcols], buf, sem).wait()

    def accum(glidx, buf):
        for g16 in range(GB // 16):
            lv = glidx[pl.ds(pl.multiple_of(g16 * 16, 16), 16)]
            for i in range(16):
                li = lv[i]
                ab = pl.multiple_of(li * D, 16)
                bi = g16 * 16 + i
                for q in range(D // 32):
                    qo = pl.multiple_of(q * 16, 16)
                    v32 = buf[bi, pl.ds(qo, 16)]
                    pab = plsc.bitcast(v32, jnp.bfloat16)
                    va, vb = plsc.unpack(
                        pab, format=plsc.PackFormat.INTERLEAVED)
                    co = pl.multiple_of(q * 32, 32)
                    plsc.addupdate(acc.at[pl.ds(ab + co, 16)], va)
                    plsc.addupdate(acc.at[pl.ds(ab + co + 16, 16)], vb)

    def drain(cnt):
        for p in range(1):
            ppos = jnp.full((16,), cnt + p * 16, jnp.int32) + iota16
            plsc.store_scatter(comp_p, [ppos], trash_p)
        nb = (cnt + GB - 1) // GB

        @pl.when(nb > 0)
        def _():
            g_start(0, gc0, gl0, buf0, sem0)

        def pair(g2, carry):
            b0 = g2 * 2
            b1 = b0 + 1

            @pl.when(b0 < nb)
            def _():
                g_wait(gc0, buf0, sem0)

                @pl.when(b1 < nb)
                def _():
                    g_start(b1, gc1, gl1, buf1, sem1)

                accum(gl0, buf0)

            @pl.when(b1 < nb)
            def _():
                g_wait(gc1, buf1, sem1)

                @pl.when(b1 + 1 < nb)
                def _():
                    g_start(b1 + 1, gc0, gl0, buf0, sem0)

                accum(gl1, buf1)

            return carry

        lax.fori_loop(0, (nb + 1) // 2, pair, 0)
        return 0

    def e_start(k, es, sem):
        koff = pl.multiple_of(k * SCC, SCC)
        pltpu.make_async_copy(
            edges_hbm.at[pl.ds(koff, SCC)], es, sem).start()

    def e_wait(k, es, sem):
        koff = pl.multiple_of(k * SCC, SCC)
        pltpu.make_async_copy(
            edges_hbm.at[pl.ds(koff, SCC)], es, sem).wait()

    def scan(es, ccs):
        def vec(i, ccs):
            jj = pl.multiple_of(i * 16, 16)
            p = es[pl.ds(jj, 16)]
            m = (p >= losh_v) & (p < hish_v)
            pcv = plsc.all_reduce_population_count(m)
            incl = plsc.cumsum(jnp.where(m, one_v, zero_v))
            pos = jnp.where(m, ccs + incl - one_v, dump_v)
            plsc.store_scatter(comp_p, [pos], p - losh_v)
            return ccs + pcv

        return lax.fori_loop(0, SCC // 16, vec, ccs, unroll=4)

    e_start(0, es0, sems)
    ccs0 = jnp.zeros((16,), jnp.int32)

    def chunk(k, ccs):
        even = k % 2 == 0

        @pl.when(even)
        def _():
            e_wait(k, es0, sems)

        @pl.when(~even)
        def _():
            e_wait(k, es1, sems)

        @pl.when(k + 1 < NSC)
        def _():
            @pl.when(even)
            def _():
                e_start(k + 1, es1, sems)

            @pl.when(~even)
            def _():
                e_start(k + 1, es0, sems)

        ccs = lax.cond(
            even,
            lambda cc: scan(es0, cc),
            lambda cc: scan(es1, cc),
            ccs,
        )
        cnt = ccs[0]
        cnt = lax.cond(
            (cnt >= DRAIN_AT) | (k == NSC - 1), drain, lambda cc: cc, cnt)
        return jnp.full((16,), cnt, jnp.int32)

    lax.fori_loop(0, NSC, chunk, ccs0)

    pltpu.sync_copy(acc.at[pl.ds(0, OUT_WORDS)],
                    agg_hbm.at[pl.ds(lo * D, OUT_WORDS)])


def _perm():
    idx = []
    for q in range(D // 32):
        for t in range(16):
            idx.append(q * 32 + t)
            idx.append(q * 32 + 16 + t)
    return idx


_PERM = tuple(_perm())


def kernel(x, edge_index, W, b):
    x_pad = jnp.concatenate(
        [x, jnp.zeros((NPAD - N, D), x.dtype)], axis=0)
    out = _matmul(x_pad, W)
    out_bf = out.astype(jnp.bfloat16)[:, jnp.array(_PERM, jnp.int32)]
    out32 = jax.lax.bitcast_convert_type(
        out_bf.reshape(NPAD, D // 2, 2), jnp.int32)
    edges = (edge_index[0] << PSHIFT) | edge_index[1]
    zeros = jnp.zeros((ACC_WORDS,), jnp.float32)
    agg = _sc_agg(out32, edges, zeros)
    return agg.reshape(NPAD, D)[:N] + b
